# baseline (speedup 1.0000x reference)
import functools

import jax
import jax.numpy as jnp
from jax import lax
from jax.experimental import pallas as pl
from jax.experimental.pallas import tpu as pltpu
from jax.experimental.pallas import tpu_sc as plsc

N_NODES = 10000
D = 128
N_EDGES = 320000
BN_EPS = 1e-5

NC = 2
NS = 16
NW = NC * NS
EW = N_EDGES // NW
CK = 80
CH = EW // CK
NBUF = 2
NPAD = 10240
RT = NPAD // NS


def _sc_aggregate(x, col_w, row_w):
    mesh = plsc.VectorSubcoreMesh(core_axis_name="c", subcore_axis_name="s")

    @functools.partial(
        pl.kernel,
        out_type=jax.ShapeDtypeStruct((NC, NPAD, D), jnp.float32),
        mesh=mesh,
        scratch_types=[
            pltpu.VMEM_SHARED((NPAD, D), jnp.float32),
            pltpu.VMEM((EW,), jnp.int32),
            pltpu.VMEM((CH, CK), jnp.int32),
            pltpu.VMEM((NBUF, CK, D), jnp.float32),
            [pltpu.SemaphoreType.DMA] * NBUF,
            [pltpu.SemaphoreType.DMA] * NBUF,
        ],
    )
    def agg_kernel(x_hbm, col_hbm, row_hbm, out_hbm, acc, cidx, ridx, rows,
                   gsems, ssems):
        core = lax.axis_index("c")
        sid = lax.axis_index("s")
        wid = sid * NC + core

        pltpu.async_copy(col_hbm.at[wid], cidx, ssems[0])
        pltpu.async_copy(row_hbm.at[wid], ridx, ssems[1])

        def _zfill(k, carry):
            rows[0, k // (D // 16), pl.ds((k % (D // 16)) * 16, 16)] = (
                jnp.zeros((16,), jnp.float32))
            return carry
        lax.fori_loop(0, CK * (D // 16), _zfill, 0)

        def _zcopy(b, carry):
            pltpu.sync_copy(rows.at[0], acc.at[pl.ds(sid * RT + b * CK, CK)])
            return carry
        lax.fori_loop(0, RT // CK, _zcopy, 0)

        pltpu.make_async_copy(col_hbm.at[wid], cidx, ssems[0]).wait()
        pltpu.make_async_copy(row_hbm.at[wid], ridx, ssems[1]).wait()
        for b in range(NBUF):
            pltpu.async_copy(x_hbm.at[cidx.at[pl.ds(b * CK, CK)]],
                             rows.at[b], gsems[b])
        plsc.subcore_barrier()

        def _round(i, carry):
            for b in range(NBUF):
                c = i * NBUF + b
                pltpu.make_async_copy(x_hbm.at[pl.ds(0, CK)], rows.at[b],
                                      gsems[b]).wait()
                pltpu.sync_copy(rows.at[b], acc.at[ridx.at[c]], add=True)

                @pl.when(c + NBUF < CH)
                def _():
                    pltpu.async_copy(
                        x_hbm.at[cidx.at[pl.ds((c + NBUF) * CK, CK)]],
                        rows.at[b], gsems[b])
            return carry
        lax.fori_loop(0, CH // NBUF, _round, 0)
        pltpu.make_async_copy(x_hbm.at[pl.ds(0, CK)], rows.at[0],
                              gsems[0]).wait()
        pltpu.sync_copy(rows.at[0], acc.at[ridx.at[CH - 1]], add=True)
        plsc.subcore_barrier()

        pltpu.sync_copy(acc.at[pl.ds(sid * RT, RT)],
                        out_hbm.at[core, pl.ds(sid * RT, RT)])

    return agg_kernel(x, col_w, row_w)


BLK = 2000
NB = N_NODES // BLK


def _mlp_body(x_hbm, p_hbm, eps_ref, w1_ref, b1_ref, g1_ref, be1_ref,
              w2_ref, b2_ref, g2_ref, be2_ref, o_hbm,
              h_ref, xb, pb, ob, semx, semp, semo):
    eps = eps_ref[0]
    w1 = w1_ref[...].astype(jnp.bfloat16)
    w2 = w2_ref[...].astype(jnp.bfloat16)

    def start_in(i):
        b = i % 2
        r = pl.ds(i * BLK, BLK)
        return (pltpu.async_copy(x_hbm.at[r], xb.at[b], semx.at[b]),
                pltpu.async_copy(p_hbm.at[0, r], pb.at[0, b], semp.at[0, b]),
                pltpu.async_copy(p_hbm.at[1, r], pb.at[1, b], semp.at[1, b]))

    pend = start_in(0)
    s1 = jnp.zeros((1, D), jnp.float32)
    s2 = jnp.zeros((1, D), jnp.float32)
    for i in range(NB):
        b = i % 2
        nxt = start_in(i + 1) if i + 1 < NB else None
        for d in pend:
            d.wait()
        out = (1.0 + eps) * xb[b] + pb[0, b] + pb[1, b]
        h = lax.dot_general(out.astype(jnp.bfloat16), w1,
                            (((1,), (1,)), ((), ())),
                            preferred_element_type=jnp.float32)
        h = h + b1_ref[...]
        h_ref[pl.ds(i * BLK, BLK), :] = h
        s1 = s1 + jnp.sum(h, axis=0, keepdims=True)
        s2 = s2 + jnp.sum(h * h, axis=0, keepdims=True)
        pend = nxt

    mean = s1 * (1.0 / N_NODES)
    var = s2 * (1.0 / N_NODES) - mean * mean
    a1 = g1_ref[...] * lax.rsqrt(var + BN_EPS)
    c1 = be1_ref[...] - mean * a1

    s1 = jnp.zeros((1, D), jnp.float32)
    s2 = jnp.zeros((1, D), jnp.float32)
    for i in range(NB):
        r = pl.ds(i * BLK, BLK)
        h = jnp.maximum(h_ref[r, :] * a1 + c1, 0.0)
        h = lax.dot_general(h.astype(jnp.bfloat16), w2,
                            (((1,), (1,)), ((), ())),
                            preferred_element_type=jnp.float32)
        h = h + b2_ref[...]
        h_ref[r, :] = h
        s1 = s1 + jnp.sum(h, axis=0, keepdims=True)
        s2 = s2 + jnp.sum(h * h, axis=0, keepdims=True)

    mean = s1 * (1.0 / N_NODES)
    var = s2 * (1.0 / N_NODES) - mean * mean
    a2 = g2_ref[...] * lax.rsqrt(var + BN_EPS)
    c2 = be2_ref[...] - mean * a2

    dob = [None, None]
    for i in range(NB):
        b = i % 2
        r = pl.ds(i * BLK, BLK)
        if dob[b] is not None:
            dob[b].wait()
        ob[b] = jnp.maximum(h_ref[r, :] * a2 + c2, 0.0)
        dob[b] = pltpu.async_copy(ob.at[b], o_hbm.at[r], semo.at[b])
    for d in dob:
        d.wait()


def _mlp(x, partials, eps, W1, b1, g1, be1, W2, b2, g2, be2):
    vmem = pl.BlockSpec(memory_space=pltpu.VMEM)
    anym = pl.BlockSpec(memory_space=pltpu.MemorySpace.HBM)
    smem = pl.BlockSpec(memory_space=pltpu.SMEM)
    return pl.pallas_call(
        _mlp_body,
        in_specs=[anym, anym, smem] + [vmem] * 8,
        out_specs=anym,
        out_shape=jax.ShapeDtypeStruct((N_NODES, D), jnp.float32),
        scratch_shapes=[
            pltpu.VMEM((N_NODES, D), jnp.float32),
            pltpu.VMEM((2, BLK, D), jnp.float32),
            pltpu.VMEM((2, 2, BLK, D), jnp.float32),
            pltpu.VMEM((2, BLK, D), jnp.float32),
            pltpu.SemaphoreType.DMA((2,)),
            pltpu.SemaphoreType.DMA((2, 2)),
            pltpu.SemaphoreType.DMA((2,)),
        ],
    )(x, partials, eps, W1, b1.reshape(1, D), g1.reshape(1, D),
      be1.reshape(1, D), W2, b2.reshape(1, D), g2.reshape(1, D),
      be2.reshape(1, D))


def kernel(x, edge_index, eps, W1, b1, g1, be1, W2, b2, g2, be2):
    row = edge_index[0].astype(jnp.int32).reshape(NW, CH, CK)
    col = edge_index[1].astype(jnp.int32).reshape(NW, EW)
    partials = _sc_aggregate(x, col, row)
    return _mlp(x, partials, eps, W1, b1, g1, be1, W2, b2, g2, be2)


# speedup vs baseline: 1.0127x; 1.0127x over previous
import functools

import jax
import jax.numpy as jnp
from jax import lax
from jax.experimental import pallas as pl
from jax.experimental.pallas import tpu as pltpu
from jax.experimental.pallas import tpu_sc as plsc

N_NODES = 10000
D = 128
N_EDGES = 320000
BN_EPS = 1e-5

NC = 2
NS = 16
NW = NC * NS
EW = N_EDGES // NW
CK = 80
CH = EW // CK
NBUF = 2
NPAD = 10240
RT = NPAD // NS


def _sc_aggregate(x, col_w, row_w):
    mesh = plsc.VectorSubcoreMesh(core_axis_name="c", subcore_axis_name="s")

    @functools.partial(
        pl.kernel,
        out_type=jax.ShapeDtypeStruct((NC, NPAD, D), jnp.float32),
        mesh=mesh,
        scratch_types=[
            pltpu.VMEM_SHARED((NPAD, D), jnp.float32),
            pltpu.VMEM((EW,), jnp.int32),
            pltpu.VMEM((CH, CK), jnp.int32),
            pltpu.VMEM((NBUF, CK, D), jnp.float32),
            [pltpu.SemaphoreType.DMA] * NBUF,
            [pltpu.SemaphoreType.DMA] * NBUF,
        ],
    )
    def agg_kernel(x_hbm, col_hbm, row_hbm, out_hbm, acc, cidx, ridx, rows,
                   gsems, ssems):
        core = lax.axis_index("c")
        sid = lax.axis_index("s")
        wid = sid * NC + core

        pltpu.async_copy(col_hbm.at[wid], cidx, ssems[0])
        pltpu.async_copy(row_hbm.at[wid], ridx, ssems[1])

        def _zfill(k, carry):
            for j in range(D // 16):
                rows[0, k, pl.ds(j * 16, 16)] = jnp.zeros((16,), jnp.float32)
            return carry
        lax.fori_loop(0, CK, _zfill, 0)

        def _zcopy(b, carry):
            pltpu.sync_copy(rows.at[0], acc.at[pl.ds(sid * RT + b * CK, CK)])
            return carry
        lax.fori_loop(0, RT // CK, _zcopy, 0)

        pltpu.make_async_copy(col_hbm.at[wid], cidx, ssems[0]).wait()
        pltpu.make_async_copy(row_hbm.at[wid], ridx, ssems[1]).wait()
        for b in range(NBUF):
            pltpu.async_copy(x_hbm.at[cidx.at[pl.ds(b * CK, CK)]],
                             rows.at[b], gsems[b])
        plsc.subcore_barrier()

        def _round(i, carry):
            for b in range(NBUF):
                c = i * NBUF + b
                pltpu.make_async_copy(x_hbm.at[pl.ds(0, CK)], rows.at[b],
                                      gsems[b]).wait()
                pltpu.sync_copy(rows.at[b], acc.at[ridx.at[c]], add=True)

                @pl.when(c + NBUF < CH)
                def _():
                    pltpu.async_copy(
                        x_hbm.at[cidx.at[pl.ds((c + NBUF) * CK, CK)]],
                        rows.at[b], gsems[b])
            return carry
        lax.fori_loop(0, CH // NBUF, _round, 0)
        pltpu.make_async_copy(x_hbm.at[pl.ds(0, CK)], rows.at[0],
                              gsems[0]).wait()
        pltpu.sync_copy(rows.at[0], acc.at[ridx.at[CH - 1]], add=True)
        plsc.subcore_barrier()

        pltpu.sync_copy(acc.at[pl.ds(sid * RT, RT)],
                        out_hbm.at[core, pl.ds(sid * RT, RT)])

    return agg_kernel(x, col_w, row_w)


BLK = 2000
NB = N_NODES // BLK


def _mlp_body(x_hbm, p_hbm, eps_ref, w1_ref, b1_ref, g1_ref, be1_ref,
              w2_ref, b2_ref, g2_ref, be2_ref, o_hbm,
              h_ref, xb, pb, ob, semx, semp, semo):
    eps = eps_ref[0]
    w1 = w1_ref[...].astype(jnp.bfloat16)
    w2 = w2_ref[...].astype(jnp.bfloat16)

    def start_in(i):
        b = i % 2
        r = pl.ds(i * BLK, BLK)
        return (pltpu.async_copy(x_hbm.at[r], xb.at[b], semx.at[b]),
                pltpu.async_copy(p_hbm.at[0, r], pb.at[0, b], semp.at[0, b]),
                pltpu.async_copy(p_hbm.at[1, r], pb.at[1, b], semp.at[1, b]))

    pend = start_in(0)
    s1 = jnp.zeros((1, D), jnp.float32)
    s2 = jnp.zeros((1, D), jnp.float32)
    for i in range(NB):
        b = i % 2
        nxt = start_in(i + 1) if i + 1 < NB else None
        for d in pend:
            d.wait()
        out = (1.0 + eps) * xb[b] + pb[0, b] + pb[1, b]
        h = lax.dot_general(out.astype(jnp.bfloat16), w1,
                            (((1,), (1,)), ((), ())),
                            preferred_element_type=jnp.float32)
        h = h + b1_ref[...]
        h_ref[pl.ds(i * BLK, BLK), :] = h
        s1 = s1 + jnp.sum(h, axis=0, keepdims=True)
        s2 = s2 + jnp.sum(h * h, axis=0, keepdims=True)
        pend = nxt

    mean = s1 * (1.0 / N_NODES)
    var = s2 * (1.0 / N_NODES) - mean * mean
    a1 = g1_ref[...] * lax.rsqrt(var + BN_EPS)
    c1 = be1_ref[...] - mean * a1

    s1 = jnp.zeros((1, D), jnp.float32)
    s2 = jnp.zeros((1, D), jnp.float32)
    for i in range(NB):
        r = pl.ds(i * BLK, BLK)
        h = jnp.maximum(h_ref[r, :] * a1 + c1, 0.0)
        h = lax.dot_general(h.astype(jnp.bfloat16), w2,
                            (((1,), (1,)), ((), ())),
                            preferred_element_type=jnp.float32)
        h = h + b2_ref[...]
        h_ref[r, :] = h
        s1 = s1 + jnp.sum(h, axis=0, keepdims=True)
        s2 = s2 + jnp.sum(h * h, axis=0, keepdims=True)

    mean = s1 * (1.0 / N_NODES)
    var = s2 * (1.0 / N_NODES) - mean * mean
    a2 = g2_ref[...] * lax.rsqrt(var + BN_EPS)
    c2 = be2_ref[...] - mean * a2

    dob = [None, None]
    for i in range(NB):
        b = i % 2
        r = pl.ds(i * BLK, BLK)
        if dob[b] is not None:
            dob[b].wait()
        ob[b] = jnp.maximum(h_ref[r, :] * a2 + c2, 0.0)
        dob[b] = pltpu.async_copy(ob.at[b], o_hbm.at[r], semo.at[b])
    for d in dob:
        d.wait()


def _mlp(x, partials, eps, W1, b1, g1, be1, W2, b2, g2, be2):
    vmem = pl.BlockSpec(memory_space=pltpu.VMEM)
    anym = pl.BlockSpec(memory_space=pltpu.MemorySpace.HBM)
    smem = pl.BlockSpec(memory_space=pltpu.SMEM)
    return pl.pallas_call(
        _mlp_body,
        in_specs=[anym, anym, smem] + [vmem] * 8,
        out_specs=anym,
        out_shape=jax.ShapeDtypeStruct((N_NODES, D), jnp.float32),
        scratch_shapes=[
            pltpu.VMEM((N_NODES, D), jnp.float32),
            pltpu.VMEM((2, BLK, D), jnp.float32),
            pltpu.VMEM((2, 2, BLK, D), jnp.float32),
            pltpu.VMEM((2, BLK, D), jnp.float32),
            pltpu.SemaphoreType.DMA((2,)),
            pltpu.SemaphoreType.DMA((2, 2)),
            pltpu.SemaphoreType.DMA((2,)),
        ],
    )(x, partials, eps, W1, b1.reshape(1, D), g1.reshape(1, D),
      be1.reshape(1, D), W2, b2.reshape(1, D), g2.reshape(1, D),
      be2.reshape(1, D))


def kernel(x, edge_index, eps, W1, b1, g1, be1, W2, b2, g2, be2):
    row = edge_index[0].astype(jnp.int32).reshape(NW, CH, CK)
    col = edge_index[1].astype(jnp.int32).reshape(NW, EW)
    partials = _sc_aggregate(x, col, row)
    return _mlp(x, partials, eps, W1, b1, g1, be1, W2, b2, g2, be2)
